# baseline (speedup 1.0000x reference)
import functools

import jax
import jax.numpy as jnp
from jax import lax
from jax.experimental import pallas as pl
from jax.experimental.pallas import tpu as pltpu
from jax.experimental.pallas import tpu_sc as plsc

N1 = 320000
D = 128
NNZ = 1280000

NC = 2
NS = 16
R = 12800
NRANGES = N1 // R
CHUNK = NNZ // NS
E = 640
NB = CHUNK // E
FC = 128
LISTCAP = 896
ROWS_PER_TILE = R // NS
ZROWS = 25

BM = 2560


def _sc_spmm_body(rows_hbm, cols_hbm, vals_hbm, x_hbm, out_hbm,
                  meta_r, meta_c, meta_v, fl_r, fl_c, fl_v,
                  sidx, cidx, gbuf, zbuf, acc, sem):
    c = lax.axis_index("c")
    s = lax.axis_index("s")
    tile_base = s * CHUNK

    def zero_zbuf(i, _):
        q = i % 8
        row = i // 8
        zbuf[row, pl.ds(q * 16, 16)] = jnp.zeros((16,), jnp.float32)
        return 0

    lax.fori_loop(0, ZROWS * 8, zero_zbuf, 0)

    def flush_chunk(j, _):
        def stage(t, _):
            sidx[pl.ds(16 * t, 16)] = fl_r[pl.ds(FC * j + 16 * t, 16)]
            cidx[pl.ds(16 * t, 16)] = fl_c[pl.ds(FC * j + 16 * t, 16)]
            return 0

        lax.fori_loop(0, FC // 16, stage, 0)
        pltpu.async_copy(x_hbm.at[cidx], gbuf, sem).wait()

        def scale_group(t, _):
            vv = fl_v[pl.ds(FC * j + 16 * t, 16)]
            for e2 in range(16):
                bv = jnp.full((16,), vv[e2], jnp.float32)
                e = 16 * t + e2
                for q in range(8):
                    gbuf[e, pl.ds(16 * q, 16)] = (
                        gbuf[e, pl.ds(16 * q, 16)] * bv)
            return 0

        lax.fori_loop(0, FC // 16, scale_group, 0)
        pltpu.sync_copy(gbuf, acc.at[sidx], add=True)
        return 0

    def drain(n):
        k = n // FC
        lax.fori_loop(0, k, flush_chunk, 0)
        rem = FC * k

        def move(t, _):
            fl_r[pl.ds(16 * t, 16)] = fl_r[pl.ds(rem + 16 * t, 16)]
            fl_c[pl.ds(16 * t, 16)] = fl_c[pl.ds(rem + 16 * t, 16)]
            fl_v[pl.ds(16 * t, 16)] = fl_v[pl.ds(rem + 16 * t, 16)]
            return 0

        lax.fori_loop(0, FC // 16, move, 0)
        return n - rem

    def run_pass(p, _):
        rng = 2 * p + c
        lo = rng * R
        hi = lo + R
        for rr in range(ROWS_PER_TILE // ZROWS):
            pltpu.sync_copy(
                zbuf, acc.at[pl.ds(s * ROWS_PER_TILE + rr * ZROWS, ZROWS)])
        plsc.subcore_barrier()

        def block(b, n):
            base = tile_base + b * E
            pltpu.sync_copy(rows_hbm.at[pl.ds(base, E)], meta_r)
            pltpu.sync_copy(cols_hbm.at[pl.ds(base, E)], meta_c)
            pltpu.sync_copy(vals_hbm.at[pl.ds(base, E)], meta_v)

            def vreg(v, n):
                r = meta_r[pl.ds(v * 16, 16)]
                m = (r >= lo) & (r < hi)
                cnt = plsc.all_reduce_population_count(m)[0]
                plsc.store_compressed(fl_r.at[pl.ds(n, 16)], r - lo, mask=m)
                plsc.store_compressed(fl_c.at[pl.ds(n, 16)],
                                      meta_c[pl.ds(v * 16, 16)], mask=m)
                plsc.store_compressed(fl_v.at[pl.ds(n, 16)],
                                      meta_v[pl.ds(v * 16, 16)], mask=m)
                return n + cnt

            n = lax.fori_loop(0, E // 16, vreg, n)
            return drain(n)

        n = lax.fori_loop(0, NB, block, jnp.int32(0))

        def pad(t, _):
            z = jnp.zeros((16,), jnp.int32)
            fl_r[pl.ds(n + 16 * t, 16)] = z
            fl_c[pl.ds(n + 16 * t, 16)] = z
            fl_v[pl.ds(n + 16 * t, 16)] = jnp.zeros((16,), jnp.float32)
            return 0

        lax.fori_loop(0, FC // 16, pad, 0)
        k_final = (n + FC - 1) // FC
        lax.fori_loop(0, k_final, flush_chunk, 0)

        plsc.subcore_barrier()
        pltpu.sync_copy(
            acc.at[pl.ds(s * ROWS_PER_TILE, ROWS_PER_TILE)],
            out_hbm.at[pl.ds(lo + s * ROWS_PER_TILE, ROWS_PER_TILE)])
        return 0

    np_c = jnp.where(c == 0, (NRANGES + 1) // 2, NRANGES // 2)
    lax.fori_loop(0, np_c, run_pass, 0)


@functools.partial(
    pl.kernel,
    out_type=jax.ShapeDtypeStruct((N1, D), jnp.float32),
    mesh=plsc.VectorSubcoreMesh(core_axis_name="c", subcore_axis_name="s"),
    compiler_params=pltpu.CompilerParams(needs_layout_passes=False),
    scratch_types=[
        pltpu.VMEM((E,), jnp.int32),
        pltpu.VMEM((E,), jnp.int32),
        pltpu.VMEM((E,), jnp.float32),
        pltpu.VMEM((LISTCAP,), jnp.int32),
        pltpu.VMEM((LISTCAP,), jnp.int32),
        pltpu.VMEM((LISTCAP,), jnp.float32),
        pltpu.VMEM((FC,), jnp.int32),
        pltpu.VMEM((FC,), jnp.int32),
        pltpu.VMEM((FC, D), jnp.float32),
        pltpu.VMEM((ZROWS, D), jnp.float32),
        pltpu.MemorySpace.VMEM_SHARED((R, D), jnp.float32),
        pltpu.SemaphoreType.DMA,
    ],
)
def _sc_spmm(rows_hbm, cols_hbm, vals_hbm, x_hbm, out_hbm, *scratch):
    _sc_spmm_body(rows_hbm, cols_hbm, vals_hbm, x_hbm, out_hbm, *scratch)


def _epilogue_body(s1_ref, s2_ref, x_ref, w_a_ref, w_b_ref, w_skip_ref,
                   g_ref, b_ref, out_ref):
    acc = jnp.dot(s1_ref[...], w_a_ref[...].T, preferred_element_type=jnp.float32)
    acc += jnp.dot(s2_ref[...], w_b_ref[...].T, preferred_element_type=jnp.float32)
    acc += jnp.dot(x_ref[...], w_skip_ref[...].T, preferred_element_type=jnp.float32)
    m = jnp.mean(acc, axis=-1, keepdims=True)
    cen = acc - m
    v = jnp.mean(cen * cen, axis=-1, keepdims=True)
    y = cen * jax.lax.rsqrt(v + 1e-5) * g_ref[...] + b_ref[...]
    out_ref[...] = 0.5 * y * (1.0 + jax.lax.erf(y * 0.7071067811865476))


def _epilogue(s1, s2, x, w_a, w_b, w_skip, g, b):
    grid = (N1 // BM,)
    blk = pl.BlockSpec((BM, D), lambda i: (i, 0))
    wblk = pl.BlockSpec((D, D), lambda i: (0, 0))
    vblk = pl.BlockSpec((1, D), lambda i: (0, 0))
    return pl.pallas_call(
        _epilogue_body,
        grid=grid,
        in_specs=[blk, blk, blk, wblk, wblk, wblk, vblk, vblk],
        out_specs=blk,
        out_shape=jax.ShapeDtypeStruct((N1, D), jnp.float32),
    )(s1, s2, x, w_a, w_b, w_skip, g.reshape(1, D), b.reshape(1, D))


def kernel(X_equ, X_inv, W1, W2, W3, W4, W5, W6, g_e, b_e, g_i, b_i,
           vals_Le, vals_ie, vals_Li, vals_ei,
           rows_Le, cols_Le, rows_ie, cols_ie, rows_Li, cols_Li,
           rows_ei, cols_ei):
    s_Le = _sc_spmm(rows_Le, cols_Le, vals_Le, X_equ)
    s_ie = _sc_spmm(rows_ie, cols_ie, vals_ie, X_inv)
    s_Li = _sc_spmm(rows_Li, cols_Li, vals_Li, X_inv)
    s_ei = _sc_spmm(rows_ei, cols_ei, vals_ei, X_equ)
    out_equ = _epilogue(s_Le, s_ie, X_equ, W1, W2, W5, g_e, b_e)
    out_inv = _epilogue(s_Li, s_ei, X_inv, W3, W4, W6, g_i, b_i)
    return (out_equ, out_inv)


# speedup vs baseline: 1.7435x; 1.7435x over previous
import functools

import jax
import jax.numpy as jnp
from jax import lax
from jax.experimental import pallas as pl
from jax.experimental.pallas import tpu as pltpu
from jax.experimental.pallas import tpu_sc as plsc

N1 = 320000
D = 128
NNZ = 1280000

NC = 2
NS = 16
R = 12800
NRANGES = N1 // R
CHUNK = NNZ // NS
E = 640
NB = CHUNK // E
FC = 128
LISTCAP = 896
ROWS_PER_TILE = R // NS
ZROWS = 25

BM = 2560


def _sc_spmm_body(packed_hbm, x_hbm, out_hbm,
                  meta, fl_r, fl_c, fl_v,
                  sidx, cidx, gbuf, zbuf, acc, sem, sem_a, sem_b):
    c = lax.axis_index("c")
    s = lax.axis_index("s")
    tile_blk0 = s * NB

    def zero_zbuf(i, _):
        q = i % 8
        row = i // 8
        zbuf[row, pl.ds(q * 16, 16)] = jnp.zeros((16,), jnp.float32)
        return 0

    lax.fori_loop(0, ZROWS * 8, zero_zbuf, 0)

    def flush_chunk(j, _):
        def stage(t, _):
            sidx[pl.ds(16 * t, 16)] = fl_r[pl.ds(FC * j + 16 * t, 16)]
            cidx[pl.ds(16 * t, 16)] = fl_c[pl.ds(FC * j + 16 * t, 16)]
            return 0

        lax.fori_loop(0, FC // 16, stage, 0)
        pltpu.async_copy(x_hbm.at[cidx], gbuf, sem).wait()

        def scale_group(t, _):
            vv = fl_v[pl.ds(FC * j + 16 * t, 16)]
            for e2 in range(16):
                bv = jnp.full((16,), vv[e2], jnp.float32)
                e = 16 * t + e2
                for q in range(8):
                    gbuf[e, pl.ds(16 * q, 16)] = (
                        gbuf[e, pl.ds(16 * q, 16)] * bv)
            return 0

        lax.fori_loop(0, FC // 16, scale_group, 0)
        pltpu.sync_copy(gbuf, acc.at[sidx], add=True)
        return 0

    def drain(n):
        k = n // FC
        lax.fori_loop(0, k, flush_chunk, 0)
        rem = FC * k

        @pl.when(k > 0)
        def _():
            def move(t, _):
                fl_r[pl.ds(16 * t, 16)] = fl_r[pl.ds(rem + 16 * t, 16)]
                fl_c[pl.ds(16 * t, 16)] = fl_c[pl.ds(rem + 16 * t, 16)]
                fl_v[pl.ds(16 * t, 16)] = fl_v[pl.ds(rem + 16 * t, 16)]
                return 0

            lax.fori_loop(0, FC // 16, move, 0)
        return n - rem

    def run_pass(p, _):
        rng = 2 * p + c
        lo = rng * R
        hi = lo + R
        for rr in range(ROWS_PER_TILE // ZROWS):
            pltpu.sync_copy(
                zbuf, acc.at[pl.ds(s * ROWS_PER_TILE + rr * ZROWS, ZROWS)])
        plsc.subcore_barrier()

        pltpu.async_copy(packed_hbm.at[tile_blk0], meta.at[0], sem_a)

        def block(b, n):
            buf = b % 2

            @pl.when(buf == 0)
            def _():
                pltpu.make_async_copy(
                    packed_hbm.at[tile_blk0], meta.at[0], sem_a).wait()

                @pl.when(b + 1 < NB)
                def _():
                    pltpu.async_copy(packed_hbm.at[tile_blk0 + b + 1],
                                     meta.at[1], sem_b)

            @pl.when(buf == 1)
            def _():
                pltpu.make_async_copy(
                    packed_hbm.at[tile_blk0], meta.at[1], sem_b).wait()

                @pl.when(b + 1 < NB)
                def _():
                    pltpu.async_copy(packed_hbm.at[tile_blk0 + b + 1],
                                     meta.at[0], sem_a)

            def vreg(v, n):
                r = meta[buf, pl.ds(v * 16, 16)]
                m = (r >= lo) & (r < hi)
                cnt = plsc.all_reduce_population_count(m)[0]
                plsc.store_compressed(fl_r.at[pl.ds(n, 16)], r - lo, mask=m)
                plsc.store_compressed(fl_c.at[pl.ds(n, 16)],
                                      meta[buf, pl.ds(E + v * 16, 16)], mask=m)
                plsc.store_compressed(
                    fl_v.at[pl.ds(n, 16)],
                    plsc.bitcast(meta[buf, pl.ds(2 * E + v * 16, 16)],
                                 jnp.float32), mask=m)
                return n + cnt

            n = lax.fori_loop(0, E // 16, vreg, n, unroll=4)
            return drain(n)

        n = lax.fori_loop(0, NB, block, jnp.int32(0))

        def pad(t, _):
            z = jnp.zeros((16,), jnp.int32)
            fl_r[pl.ds(n + 16 * t, 16)] = z
            fl_c[pl.ds(n + 16 * t, 16)] = z
            fl_v[pl.ds(n + 16 * t, 16)] = jnp.zeros((16,), jnp.float32)
            return 0

        lax.fori_loop(0, FC // 16, pad, 0)
        k_final = (n + FC - 1) // FC
        lax.fori_loop(0, k_final, flush_chunk, 0)

        plsc.subcore_barrier()
        pltpu.sync_copy(
            acc.at[pl.ds(s * ROWS_PER_TILE, ROWS_PER_TILE)],
            out_hbm.at[pl.ds(lo + s * ROWS_PER_TILE, ROWS_PER_TILE)])
        return 0

    np_c = jnp.where(c == 0, (NRANGES + 1) // 2, NRANGES // 2)
    lax.fori_loop(0, np_c, run_pass, 0)


@functools.partial(
    pl.kernel,
    out_type=jax.ShapeDtypeStruct((N1, D), jnp.float32),
    mesh=plsc.VectorSubcoreMesh(core_axis_name="c", subcore_axis_name="s"),
    compiler_params=pltpu.CompilerParams(needs_layout_passes=False),
    scratch_types=[
        pltpu.VMEM((2, 3 * E), jnp.int32),
        pltpu.VMEM((LISTCAP,), jnp.int32),
        pltpu.VMEM((LISTCAP,), jnp.int32),
        pltpu.VMEM((LISTCAP,), jnp.float32),
        pltpu.VMEM((FC,), jnp.int32),
        pltpu.VMEM((FC,), jnp.int32),
        pltpu.VMEM((FC, D), jnp.float32),
        pltpu.VMEM((ZROWS, D), jnp.float32),
        pltpu.MemorySpace.VMEM_SHARED((R, D), jnp.float32),
        pltpu.SemaphoreType.DMA,
        pltpu.SemaphoreType.DMA,
        pltpu.SemaphoreType.DMA,
    ],
)
def _sc_spmm(packed_hbm, x_hbm, out_hbm, *scratch):
    _sc_spmm_body(packed_hbm, x_hbm, out_hbm, *scratch)


def _pack_meta(rows, cols, vals):
    nblk = NNZ // E
    vi = lax.bitcast_convert_type(vals, jnp.int32)
    return jnp.stack([rows.reshape(nblk, E), cols.reshape(nblk, E),
                      vi.reshape(nblk, E)], axis=1).reshape(nblk, 3 * E)


def _epilogue_body(s1_ref, s2_ref, x_ref, w_a_ref, w_b_ref, w_skip_ref,
                   g_ref, b_ref, out_ref):
    acc = jnp.dot(s1_ref[...], w_a_ref[...].T, preferred_element_type=jnp.float32)
    acc += jnp.dot(s2_ref[...], w_b_ref[...].T, preferred_element_type=jnp.float32)
    acc += jnp.dot(x_ref[...], w_skip_ref[...].T, preferred_element_type=jnp.float32)
    m = jnp.mean(acc, axis=-1, keepdims=True)
    cen = acc - m
    v = jnp.mean(cen * cen, axis=-1, keepdims=True)
    y = cen * jax.lax.rsqrt(v + 1e-5) * g_ref[...] + b_ref[...]
    out_ref[...] = 0.5 * y * (1.0 + jax.lax.erf(y * 0.7071067811865476))


def _epilogue(s1, s2, x, w_a, w_b, w_skip, g, b):
    grid = (N1 // BM,)
    blk = pl.BlockSpec((BM, D), lambda i: (i, 0))
    wblk = pl.BlockSpec((D, D), lambda i: (0, 0))
    vblk = pl.BlockSpec((1, D), lambda i: (0, 0))
    return pl.pallas_call(
        _epilogue_body,
        grid=grid,
        in_specs=[blk, blk, blk, wblk, wblk, wblk, vblk, vblk],
        out_specs=blk,
        out_shape=jax.ShapeDtypeStruct((N1, D), jnp.float32),
    )(s1, s2, x, w_a, w_b, w_skip, g.reshape(1, D), b.reshape(1, D))


def kernel(X_equ, X_inv, W1, W2, W3, W4, W5, W6, g_e, b_e, g_i, b_i,
           vals_Le, vals_ie, vals_Li, vals_ei,
           rows_Le, cols_Le, rows_ie, cols_ie, rows_Li, cols_Li,
           rows_ei, cols_ei):
    s_Le = _sc_spmm(_pack_meta(rows_Le, cols_Le, vals_Le), X_equ)
    s_ie = _sc_spmm(_pack_meta(rows_ie, cols_ie, vals_ie), X_inv)
    s_Li = _sc_spmm(_pack_meta(rows_Li, cols_Li, vals_Li), X_inv)
    s_ei = _sc_spmm(_pack_meta(rows_ei, cols_ei, vals_ei), X_equ)
    out_equ = _epilogue(s_Le, s_ie, X_equ, W1, W2, W5, g_e, b_e)
    out_inv = _epilogue(s_Li, s_ei, X_inv, W3, W4, W6, g_i, b_i)
    return (out_equ, out_inv)


# speedup vs baseline: 2.4276x; 1.3924x over previous
import functools

import jax
import jax.numpy as jnp
from jax import lax
from jax.experimental import pallas as pl
from jax.experimental.pallas import tpu as pltpu
from jax.experimental.pallas import tpu_sc as plsc

N1 = 320000
D = 128
NNZ = 1280000

NC = 2
NS = 16
R = 12800
NRANGES = N1 // R
CHUNK = NNZ // NS
E = 640
NB = CHUNK // E
FC = 64
LISTCAP = 768
ROWS_PER_TILE = R // NS
ZROWS = 25

BM = 2560


def _sc_spmm_body(packed_hbm, x_hbm, out_hbm,
                  meta, fl_r, fl_c, fl_v,
                  sidx, cidx, vidx, gbuf, zbuf, acc,
                  semg0, semg1, sem_a, sem_b):
    c = lax.axis_index("c")
    s = lax.axis_index("s")
    tile_blk0 = s * NB

    def zero_zbuf(i, _):
        q = i % 8
        row = i // 8
        zbuf[row, pl.ds(q * 16, 16)] = jnp.zeros((16,), jnp.float32)
        return 0

    lax.fori_loop(0, ZROWS * 8, zero_zbuf, 0)


    def stage_issue(j, g):
        def stage(t, _):
            sidx[g, pl.ds(16 * t, 16)] = fl_r[pl.ds(FC * j + 16 * t, 16)]
            cidx[g, pl.ds(16 * t, 16)] = fl_c[pl.ds(FC * j + 16 * t, 16)]
            vidx[g, pl.ds(16 * t, 16)] = fl_v[pl.ds(FC * j + 16 * t, 16)]
            return 0

        lax.fori_loop(0, FC // 16, stage, 0)
        pltpu.async_copy(x_hbm.at[cidx.at[g]], gbuf.at[g],
                         semg0 if g == 0 else semg1)

    def complete(g):
        pltpu.make_async_copy(x_hbm.at[cidx.at[g]], gbuf.at[g],
                              semg0 if g == 0 else semg1).wait()

        def scale_group(t, _):
            vv = vidx[g, pl.ds(16 * t, 16)]
            for e2 in range(16):
                bv = jnp.full((16,), vv[e2], jnp.float32)
                e = 16 * t + e2
                for q in range(8):
                    gbuf[g, e, pl.ds(16 * q, 16)] = (
                        gbuf[g, e, pl.ds(16 * q, 16)] * bv)
            return 0

        lax.fori_loop(0, FC // 16, scale_group, 0)
        pltpu.sync_copy(gbuf.at[g], acc.at[sidx.at[g]], add=True)

    def complete_dyn(gd):
        @pl.when(gd == 0)
        def _():
            complete(0)

        @pl.when(gd == 1)
        def _():
            complete(1)

    def stage_issue_dyn(j, gd):
        @pl.when(gd == 0)
        def _():
            stage_issue(j, 0)

        @pl.when(gd == 1)
        def _():
            stage_issue(j, 1)

    def drain(n, cc, pend):
        k = n // FC

        def jloop(j, carry):
            cc, pend = carry

            @pl.when(pend == 1)
            def _():
                complete_dyn((cc - 1) % 2)

            stage_issue_dyn(j, cc % 2)
            return (cc + 1, jnp.int32(1))

        cc, pend = lax.fori_loop(0, k, jloop, (cc, pend))
        rem = FC * k

        @pl.when(k > 0)
        def _():
            def move(t, _):
                fl_r[pl.ds(16 * t, 16)] = fl_r[pl.ds(rem + 16 * t, 16)]
                fl_c[pl.ds(16 * t, 16)] = fl_c[pl.ds(rem + 16 * t, 16)]
                fl_v[pl.ds(16 * t, 16)] = fl_v[pl.ds(rem + 16 * t, 16)]
                return 0

            lax.fori_loop(0, FC // 16, move, 0)

        return n - rem, cc, pend

    def run_pass(p, _):
        rng = 2 * p + c
        lo = rng * R
        hi = lo + R
        for rr in range(ROWS_PER_TILE // ZROWS):
            pltpu.sync_copy(
                zbuf, acc.at[pl.ds(s * ROWS_PER_TILE + rr * ZROWS, ZROWS)])
        plsc.subcore_barrier()

        pltpu.async_copy(packed_hbm.at[tile_blk0], meta.at[0], sem_a)

        def block(b, carry):
            n, cc, pend = carry
            buf = b % 2

            @pl.when(buf == 0)
            def _():
                pltpu.make_async_copy(
                    packed_hbm.at[tile_blk0], meta.at[0], sem_a).wait()

                @pl.when(b + 1 < NB)
                def _():
                    pltpu.async_copy(packed_hbm.at[tile_blk0 + b + 1],
                                     meta.at[1], sem_b)

            @pl.when(buf == 1)
            def _():
                pltpu.make_async_copy(
                    packed_hbm.at[tile_blk0], meta.at[1], sem_b).wait()

                @pl.when(b + 1 < NB)
                def _():
                    pltpu.async_copy(packed_hbm.at[tile_blk0 + b + 1],
                                     meta.at[0], sem_a)

            def vreg(v, n):
                r = meta[buf, pl.ds(v * 16, 16)]
                m = (r >= lo) & (r < hi)
                cnt = plsc.all_reduce_population_count(m)[0]
                plsc.store_compressed(fl_r.at[pl.ds(n, 16)], r - lo, mask=m)
                plsc.store_compressed(fl_c.at[pl.ds(n, 16)],
                                      meta[buf, pl.ds(E + v * 16, 16)], mask=m)
                plsc.store_compressed(
                    fl_v.at[pl.ds(n, 16)],
                    plsc.bitcast(meta[buf, pl.ds(2 * E + v * 16, 16)],
                                 jnp.float32), mask=m)
                return n + cnt

            n = lax.fori_loop(0, E // 16, vreg, n, unroll=8)
            return drain(n, cc, pend)

        n, cc, pend = lax.fori_loop(
            0, NB, block, (jnp.int32(0), jnp.int32(0), jnp.int32(0)))

        @pl.when(pend == 1)
        def _():
            complete_dyn((cc - 1) % 2)

        def pad(t, _):
            z = jnp.zeros((16,), jnp.int32)
            fl_r[pl.ds(n + 16 * t, 16)] = z
            fl_c[pl.ds(n + 16 * t, 16)] = z
            fl_v[pl.ds(n + 16 * t, 16)] = jnp.zeros((16,), jnp.float32)
            return 0

        lax.fori_loop(0, FC // 16, pad, 0)
        k_final = (n + FC - 1) // FC

        def jfinal(j, _):
            stage_issue(j, 0)
            complete(0)
            return 0

        lax.fori_loop(0, k_final, jfinal, 0)

        plsc.subcore_barrier()
        pltpu.sync_copy(
            acc.at[pl.ds(s * ROWS_PER_TILE, ROWS_PER_TILE)],
            out_hbm.at[pl.ds(lo + s * ROWS_PER_TILE, ROWS_PER_TILE)])
        return 0

    np_c = jnp.where(c == 0, (NRANGES + 1) // 2, NRANGES // 2)
    lax.fori_loop(0, np_c, run_pass, 0)


@functools.partial(
    pl.kernel,
    out_type=jax.ShapeDtypeStruct((N1, D), jnp.float32),
    mesh=plsc.VectorSubcoreMesh(core_axis_name="c", subcore_axis_name="s"),
    compiler_params=pltpu.CompilerParams(needs_layout_passes=False),
    scratch_types=[
        pltpu.VMEM((2, 3 * E), jnp.int32),
        pltpu.VMEM((LISTCAP,), jnp.int32),
        pltpu.VMEM((LISTCAP,), jnp.int32),
        pltpu.VMEM((LISTCAP,), jnp.float32),
        pltpu.VMEM((2, FC), jnp.int32),
        pltpu.VMEM((2, FC), jnp.int32),
        pltpu.VMEM((2, FC), jnp.float32),
        pltpu.VMEM((2, FC, D), jnp.float32),
        pltpu.VMEM((ZROWS, D), jnp.float32),
        pltpu.MemorySpace.VMEM_SHARED((R, D), jnp.float32),
        pltpu.SemaphoreType.DMA,
        pltpu.SemaphoreType.DMA,
        pltpu.SemaphoreType.DMA,
        pltpu.SemaphoreType.DMA,
    ],
)
def _sc_spmm(packed_hbm, x_hbm, out_hbm, *scratch):
    _sc_spmm_body(packed_hbm, x_hbm, out_hbm, *scratch)


def _pack_meta(rows, cols, vals):
    nblk = NNZ // E
    vi = lax.bitcast_convert_type(vals, jnp.int32)
    return jnp.stack([rows.reshape(nblk, E), cols.reshape(nblk, E),
                      vi.reshape(nblk, E)], axis=1).reshape(nblk, 3 * E)


def _epilogue_body(s1_ref, s2_ref, x_ref, w_a_ref, w_b_ref, w_skip_ref,
                   g_ref, b_ref, out_ref):
    acc = jnp.dot(s1_ref[...], w_a_ref[...].T, preferred_element_type=jnp.float32)
    acc += jnp.dot(s2_ref[...], w_b_ref[...].T, preferred_element_type=jnp.float32)
    acc += jnp.dot(x_ref[...], w_skip_ref[...].T, preferred_element_type=jnp.float32)
    m = jnp.mean(acc, axis=-1, keepdims=True)
    cen = acc - m
    v = jnp.mean(cen * cen, axis=-1, keepdims=True)
    y = cen * jax.lax.rsqrt(v + 1e-5) * g_ref[...] + b_ref[...]
    out_ref[...] = 0.5 * y * (1.0 + jax.lax.erf(y * 0.7071067811865476))


def _epilogue(s1, s2, x, w_a, w_b, w_skip, g, b):
    grid = (N1 // BM,)
    blk = pl.BlockSpec((BM, D), lambda i: (i, 0))
    wblk = pl.BlockSpec((D, D), lambda i: (0, 0))
    vblk = pl.BlockSpec((1, D), lambda i: (0, 0))
    return pl.pallas_call(
        _epilogue_body,
        grid=grid,
        in_specs=[blk, blk, blk, wblk, wblk, wblk, vblk, vblk],
        out_specs=blk,
        out_shape=jax.ShapeDtypeStruct((N1, D), jnp.float32),
    )(s1, s2, x, w_a, w_b, w_skip, g.reshape(1, D), b.reshape(1, D))


def kernel(X_equ, X_inv, W1, W2, W3, W4, W5, W6, g_e, b_e, g_i, b_i,
           vals_Le, vals_ie, vals_Li, vals_ei,
           rows_Le, cols_Le, rows_ie, cols_ie, rows_Li, cols_Li,
           rows_ei, cols_ei):
    s_Le = _sc_spmm(_pack_meta(rows_Le, cols_Le, vals_Le), X_equ)
    s_ie = _sc_spmm(_pack_meta(rows_ie, cols_ie, vals_ie), X_inv)
    s_Li = _sc_spmm(_pack_meta(rows_Li, cols_Li, vals_Li), X_inv)
    s_ei = _sc_spmm(_pack_meta(rows_ei, cols_ei, vals_ei), X_equ)
    out_equ = _epilogue(s_Le, s_ie, X_equ, W1, W2, W5, g_e, b_e)
    out_inv = _epilogue(s_Li, s_ei, X_inv, W3, W4, W6, g_i, b_i)
    return (out_equ, out_inv)


# speedup vs baseline: 2.6874x; 1.1070x over previous
import functools

import jax
import jax.numpy as jnp
from jax import lax
from jax.experimental import pallas as pl
from jax.experimental.pallas import tpu as pltpu
from jax.experimental.pallas import tpu_sc as plsc

N1 = 320000
D = 128
NNZ = 1280000

NC = 2
NS = 16
R = 12800
NRANGES = N1 // R
CHUNK = NNZ // NS
E = 640
NB = CHUNK // E
FC = 64
LISTCAP = 768
ROWS_PER_TILE = R // NS
ZROWS = 25

BM = 2560


def _sc_spmm_body(packed_hbm, x_hbm, out_hbm,
                  meta, fl_r, fl_c, fl_v, eix0, eix1,
                  sidx, cidx, vidx, gbuf, zbuf, acc,
                  semg0, semg1, sem_a, sem_b):
    c = lax.axis_index("c")
    s = lax.axis_index("s")
    tile_blk0 = s * NB

    def zero_zbuf(i, _):
        q = i % 8
        row = i // 8
        zbuf[row, pl.ds(q * 16, 16)] = jnp.zeros((16,), jnp.float32)
        return 0

    lax.fori_loop(0, ZROWS * 8, zero_zbuf, 0)


    def stage_issue(j, g):
        def stage(t, _):
            sidx[g, pl.ds(16 * t, 16)] = fl_r[pl.ds(FC * j + 16 * t, 16)]
            cidx[g, pl.ds(16 * t, 16)] = fl_c[pl.ds(FC * j + 16 * t, 16)]
            vidx[g, pl.ds(16 * t, 16)] = fl_v[pl.ds(FC * j + 16 * t, 16)]
            return 0

        lax.fori_loop(0, FC // 16, stage, 0)
        pltpu.async_copy(x_hbm.at[cidx.at[g]], gbuf.at[g],
                         semg0 if g == 0 else semg1)

    def complete(g):
        pltpu.make_async_copy(x_hbm.at[cidx.at[g]], gbuf.at[g],
                              semg0 if g == 0 else semg1).wait()

        def scale_group(t, _):
            vv = vidx[g, pl.ds(16 * t, 16)]
            for e2 in range(16):
                bv = jnp.full((16,), vv[e2], jnp.float32)
                e = 16 * t + e2
                for q in range(8):
                    gbuf[g, e, pl.ds(16 * q, 16)] = (
                        gbuf[g, e, pl.ds(16 * q, 16)] * bv)
            return 0

        lax.fori_loop(0, FC // 16, scale_group, 0)
        pltpu.sync_copy(gbuf.at[g], acc.at[sidx.at[g]], add=True)

    def complete_dyn(gd):
        @pl.when(gd == 0)
        def _():
            complete(0)

        @pl.when(gd == 1)
        def _():
            complete(1)

    def stage_issue_dyn(j, gd):
        @pl.when(gd == 0)
        def _():
            stage_issue(j, 0)

        @pl.when(gd == 1)
        def _():
            stage_issue(j, 1)

    def drain(n, cc, pend):
        k = n // FC

        def jloop(j, carry):
            cc, pend = carry

            @pl.when(pend == 1)
            def _():
                complete_dyn((cc - 1) % 2)

            stage_issue_dyn(j, cc % 2)
            return (cc + 1, jnp.int32(1))

        cc, pend = lax.fori_loop(0, k, jloop, (cc, pend))
        rem = FC * k

        @pl.when(k > 0)
        def _():
            def move(t, _):
                fl_r[pl.ds(16 * t, 16)] = fl_r[pl.ds(rem + 16 * t, 16)]
                fl_c[pl.ds(16 * t, 16)] = fl_c[pl.ds(rem + 16 * t, 16)]
                fl_v[pl.ds(16 * t, 16)] = fl_v[pl.ds(rem + 16 * t, 16)]
                return 0

            lax.fori_loop(0, FC // 16, move, 0)

        return n - rem, cc, pend

    def run_pass(p, _):
        rng = 2 * p + c
        lo = rng * R
        hi = lo + R
        for rr in range(ROWS_PER_TILE // ZROWS):
            pltpu.sync_copy(
                zbuf, acc.at[pl.ds(s * ROWS_PER_TILE + rr * ZROWS, ZROWS)])
        plsc.subcore_barrier()

        pltpu.async_copy(packed_hbm.at[tile_blk0],
                         meta.at[pl.ds(0, 3 * E)], sem_a)

        def block(b, carry):
            n, cc, pend = carry
            buf = b % 2

            @pl.when(buf == 0)
            def _():
                pltpu.make_async_copy(
                    packed_hbm.at[tile_blk0],
                    meta.at[pl.ds(0, 3 * E)], sem_a).wait()

                @pl.when(b + 1 < NB)
                def _():
                    pltpu.async_copy(packed_hbm.at[tile_blk0 + b + 1],
                                     meta.at[pl.ds(3 * E, 3 * E)], sem_b)

            @pl.when(buf == 1)
            def _():
                pltpu.make_async_copy(
                    packed_hbm.at[tile_blk0],
                    meta.at[pl.ds(3 * E, 3 * E)], sem_b).wait()

                @pl.when(b + 1 < NB)
                def _():
                    pltpu.async_copy(packed_hbm.at[tile_blk0 + b + 1],
                                     meta.at[pl.ds(0, 3 * E)], sem_a)

            iota = lax.iota(jnp.int32, 16)
            moff = buf * (3 * E)

            def vreg2(v, carry):
                ne0, ne1 = carry
                base0 = v * 16
                base1 = (E // 2) + v * 16
                r0 = meta[pl.ds(moff + base0, 16)]
                r1 = meta[pl.ds(moff + base1, 16)]
                m0 = (r0 >= lo) & (r0 < hi)
                m1 = (r1 >= lo) & (r1 < hi)
                c0 = plsc.all_reduce_population_count(m0)[0]
                c1 = plsc.all_reduce_population_count(m1)[0]
                plsc.store_compressed(eix0.at[pl.ds(ne0, 16)],
                                      iota + base0, mask=m0)
                plsc.store_compressed(eix1.at[pl.ds(ne1, 16)],
                                      iota + base1, mask=m1)
                return (ne0 + c0, ne1 + c1)

            ne0, ne1 = lax.fori_loop(
                0, E // 32, vreg2, (jnp.int32(0), jnp.int32(0)), unroll=4)
            eix0[pl.ds(ne0, 16)] = jnp.zeros((16,), jnp.int32)
            eix1[pl.ds(ne1, 16)] = jnp.zeros((16,), jnp.int32)

            def expand(eix, nbase):
                def body(t, _):
                    ei = eix[pl.ds(16 * t, 16)] + moff
                    rr = plsc.load_gather(meta, [ei])
                    cv = plsc.load_gather(meta, [ei + E])
                    vv = plsc.load_gather(meta, [ei + 2 * E])
                    fl_r[pl.ds(nbase + 16 * t, 16)] = rr - lo
                    fl_c[pl.ds(nbase + 16 * t, 16)] = cv
                    fl_v[pl.ds(nbase + 16 * t, 16)] = plsc.bitcast(
                        vv, jnp.float32)
                    return 0
                return body

            lax.fori_loop(0, (ne0 + 15) // 16, expand(eix0, n), 0)
            lax.fori_loop(0, (ne1 + 15) // 16, expand(eix1, n + ne0), 0)
            n = n + ne0 + ne1
            return drain(n, cc, pend)

        n, cc, pend = lax.fori_loop(
            0, NB, block, (jnp.int32(0), jnp.int32(0), jnp.int32(0)))

        @pl.when(pend == 1)
        def _():
            complete_dyn((cc - 1) % 2)

        def pad(t, _):
            z = jnp.zeros((16,), jnp.int32)
            fl_r[pl.ds(n + 16 * t, 16)] = z
            fl_c[pl.ds(n + 16 * t, 16)] = z
            fl_v[pl.ds(n + 16 * t, 16)] = jnp.zeros((16,), jnp.float32)
            return 0

        lax.fori_loop(0, FC // 16, pad, 0)
        k_final = (n + FC - 1) // FC

        def jfinal(j, _):
            stage_issue(j, 0)
            complete(0)
            return 0

        lax.fori_loop(0, k_final, jfinal, 0)

        plsc.subcore_barrier()
        pltpu.sync_copy(
            acc.at[pl.ds(s * ROWS_PER_TILE, ROWS_PER_TILE)],
            out_hbm.at[pl.ds(lo + s * ROWS_PER_TILE, ROWS_PER_TILE)])
        return 0

    np_c = jnp.where(c == 0, (NRANGES + 1) // 2, NRANGES // 2)
    lax.fori_loop(0, np_c, run_pass, 0)


@functools.partial(
    pl.kernel,
    out_type=jax.ShapeDtypeStruct((N1, D), jnp.float32),
    mesh=plsc.VectorSubcoreMesh(core_axis_name="c", subcore_axis_name="s"),
    compiler_params=pltpu.CompilerParams(needs_layout_passes=False),
    scratch_types=[
        pltpu.VMEM((2 * 3 * E,), jnp.int32),
        pltpu.VMEM((LISTCAP,), jnp.int32),
        pltpu.VMEM((LISTCAP,), jnp.int32),
        pltpu.VMEM((LISTCAP,), jnp.float32),
        pltpu.VMEM((352,), jnp.int32),
        pltpu.VMEM((352,), jnp.int32),
        pltpu.VMEM((2, FC), jnp.int32),
        pltpu.VMEM((2, FC), jnp.int32),
        pltpu.VMEM((2, FC), jnp.float32),
        pltpu.VMEM((2, FC, D), jnp.float32),
        pltpu.VMEM((ZROWS, D), jnp.float32),
        pltpu.MemorySpace.VMEM_SHARED((R, D), jnp.float32),
        pltpu.SemaphoreType.DMA,
        pltpu.SemaphoreType.DMA,
        pltpu.SemaphoreType.DMA,
        pltpu.SemaphoreType.DMA,
    ],
)
def _sc_spmm(packed_hbm, x_hbm, out_hbm, *scratch):
    _sc_spmm_body(packed_hbm, x_hbm, out_hbm, *scratch)


def _pack_meta(rows, cols, vals):
    nblk = NNZ // E
    vi = lax.bitcast_convert_type(vals, jnp.int32)
    return jnp.stack([rows.reshape(nblk, E), cols.reshape(nblk, E),
                      vi.reshape(nblk, E)], axis=1).reshape(nblk, 3 * E)


def _epilogue_body(s1_ref, s2_ref, x_ref, w_a_ref, w_b_ref, w_skip_ref,
                   g_ref, b_ref, out_ref):
    acc = jnp.dot(s1_ref[...], w_a_ref[...].T, preferred_element_type=jnp.float32)
    acc += jnp.dot(s2_ref[...], w_b_ref[...].T, preferred_element_type=jnp.float32)
    acc += jnp.dot(x_ref[...], w_skip_ref[...].T, preferred_element_type=jnp.float32)
    m = jnp.mean(acc, axis=-1, keepdims=True)
    cen = acc - m
    v = jnp.mean(cen * cen, axis=-1, keepdims=True)
    y = cen * jax.lax.rsqrt(v + 1e-5) * g_ref[...] + b_ref[...]
    out_ref[...] = 0.5 * y * (1.0 + jax.lax.erf(y * 0.7071067811865476))


def _epilogue(s1, s2, x, w_a, w_b, w_skip, g, b):
    grid = (N1 // BM,)
    blk = pl.BlockSpec((BM, D), lambda i: (i, 0))
    wblk = pl.BlockSpec((D, D), lambda i: (0, 0))
    vblk = pl.BlockSpec((1, D), lambda i: (0, 0))
    return pl.pallas_call(
        _epilogue_body,
        grid=grid,
        in_specs=[blk, blk, blk, wblk, wblk, wblk, vblk, vblk],
        out_specs=blk,
        out_shape=jax.ShapeDtypeStruct((N1, D), jnp.float32),
    )(s1, s2, x, w_a, w_b, w_skip, g.reshape(1, D), b.reshape(1, D))


def kernel(X_equ, X_inv, W1, W2, W3, W4, W5, W6, g_e, b_e, g_i, b_i,
           vals_Le, vals_ie, vals_Li, vals_ei,
           rows_Le, cols_Le, rows_ie, cols_ie, rows_Li, cols_Li,
           rows_ei, cols_ei):
    s_Le = _sc_spmm(_pack_meta(rows_Le, cols_Le, vals_Le), X_equ)
    s_ie = _sc_spmm(_pack_meta(rows_ie, cols_ie, vals_ie), X_inv)
    s_Li = _sc_spmm(_pack_meta(rows_Li, cols_Li, vals_Li), X_inv)
    s_ei = _sc_spmm(_pack_meta(rows_ei, cols_ei, vals_ei), X_equ)
    out_equ = _epilogue(s_Le, s_ie, X_equ, W1, W2, W5, g_e, b_e)
    out_inv = _epilogue(s_Li, s_ei, X_inv, W3, W4, W6, g_i, b_i)
    return (out_equ, out_inv)


# speedup vs baseline: 3.0145x; 1.1217x over previous
import functools

import jax
import jax.numpy as jnp
from jax import lax
from jax.experimental import pallas as pl
from jax.experimental.pallas import tpu as pltpu
from jax.experimental.pallas import tpu_sc as plsc

N1 = 320000
D = 128
NNZ = 1280000

NC = 2
NS = 16
R = 12800
NRANGES = N1 // R
CHUNK = NNZ // NS
E = 640
NB = CHUNK // E
FC = 64
LISTCAP = 768
ROWS_PER_TILE = R // NS
ZROWS = 25

BM = 2560


def _sc_spmm_body(packed_hbm, x_hbm, out_hbm,
                  meta, fl_r, fl_c, fl_v, eix0, eix1,
                  sidx, cidx, vidx, gbuf, zbuf, acc,
                  semg0, semg1, semsc0, semsc1, sem_a, sem_b):
    c = lax.axis_index("c")
    s = lax.axis_index("s")
    tile_blk0 = s * NB

    def zero_zbuf(i, _):
        q = i % 8
        row = i // 8
        zbuf[row, pl.ds(q * 16, 16)] = jnp.zeros((16,), jnp.float32)
        return 0

    lax.fori_loop(0, ZROWS * 8, zero_zbuf, 0)


    def stage_issue(j, g, ps_g):
        @pl.when(ps_g == 1)
        def _():
            pltpu.make_async_copy(gbuf.at[g], acc.at[sidx.at[g]],
                                  semsc0 if g == 0 else semsc1).wait()

        def stage(t, _):
            sidx[g, pl.ds(16 * t, 16)] = fl_r[pl.ds(FC * j + 16 * t, 16)]
            cidx[g, pl.ds(16 * t, 16)] = fl_c[pl.ds(FC * j + 16 * t, 16)]
            vidx[g, pl.ds(16 * t, 16)] = fl_v[pl.ds(FC * j + 16 * t, 16)]
            return 0

        lax.fori_loop(0, FC // 16, stage, 0)
        pltpu.async_copy(x_hbm.at[cidx.at[g]], gbuf.at[g],
                         semg0 if g == 0 else semg1)

    def complete(g):
        pltpu.make_async_copy(x_hbm.at[cidx.at[g]], gbuf.at[g],
                              semg0 if g == 0 else semg1).wait()

        def scale_group(t, _):
            vv = vidx[g, pl.ds(16 * t, 16)]
            for e2 in range(16):
                bv = jnp.full((16,), vv[e2], jnp.float32)
                e = 16 * t + e2
                for q in range(8):
                    gbuf[g, e, pl.ds(16 * q, 16)] = (
                        gbuf[g, e, pl.ds(16 * q, 16)] * bv)
            return 0

        lax.fori_loop(0, FC // 16, scale_group, 0)
        pltpu.async_copy(gbuf.at[g], acc.at[sidx.at[g]],
                         semsc0 if g == 0 else semsc1, add=True)

    def complete_dyn(gd):
        @pl.when(gd == 0)
        def _():
            complete(0)

        @pl.when(gd == 1)
        def _():
            complete(1)

    def stage_issue_dyn(j, gd, ps0, ps1):
        @pl.when(gd == 0)
        def _():
            stage_issue(j, 0, ps0)

        @pl.when(gd == 1)
        def _():
            stage_issue(j, 1, ps1)

    def drain(n, cc, pend, ps0, ps1):
        k = n // FC

        def jloop(j, carry):
            cc, pend, ps0, ps1 = carry

            @pl.when(pend == 1)
            def _():
                complete_dyn((cc - 1) % 2)

            par = cc % 2
            stage_issue_dyn(j, par, ps0, ps1)
            ps0 = jnp.where(par == 0, jnp.int32(1), ps0)
            ps1 = jnp.where(par == 1, jnp.int32(1), ps1)
            return (cc + 1, jnp.int32(1), ps0, ps1)

        cc, pend, ps0, ps1 = lax.fori_loop(0, k, jloop, (cc, pend, ps0, ps1))
        rem = FC * k

        @pl.when(k > 0)
        def _():
            def move(t, _):
                fl_r[pl.ds(16 * t, 16)] = fl_r[pl.ds(rem + 16 * t, 16)]
                fl_c[pl.ds(16 * t, 16)] = fl_c[pl.ds(rem + 16 * t, 16)]
                fl_v[pl.ds(16 * t, 16)] = fl_v[pl.ds(rem + 16 * t, 16)]
                return 0

            lax.fori_loop(0, FC // 16, move, 0)

        return n - rem, cc, pend, ps0, ps1

    def run_pass(p, _):
        rng = 2 * p + c
        lo = rng * R
        hi = lo + R
        for rr in range(ROWS_PER_TILE // ZROWS):
            pltpu.async_copy(
                zbuf, acc.at[pl.ds(s * ROWS_PER_TILE + rr * ZROWS, ZROWS)],
                semg0)
        for rr in range(ROWS_PER_TILE // ZROWS):
            pltpu.make_async_copy(
                zbuf, acc.at[pl.ds(s * ROWS_PER_TILE + rr * ZROWS, ZROWS)],
                semg0).wait()
        plsc.subcore_barrier()

        pltpu.async_copy(packed_hbm.at[tile_blk0],
                         meta.at[pl.ds(0, 3 * E)], sem_a)

        def block(b, carry):
            n, cc, pend, ps0, ps1 = carry
            buf = b % 2

            @pl.when(buf == 0)
            def _():
                pltpu.make_async_copy(
                    packed_hbm.at[tile_blk0],
                    meta.at[pl.ds(0, 3 * E)], sem_a).wait()

                @pl.when(b + 1 < NB)
                def _():
                    pltpu.async_copy(packed_hbm.at[tile_blk0 + b + 1],
                                     meta.at[pl.ds(3 * E, 3 * E)], sem_b)

            @pl.when(buf == 1)
            def _():
                pltpu.make_async_copy(
                    packed_hbm.at[tile_blk0],
                    meta.at[pl.ds(3 * E, 3 * E)], sem_b).wait()

                @pl.when(b + 1 < NB)
                def _():
                    pltpu.async_copy(packed_hbm.at[tile_blk0 + b + 1],
                                     meta.at[pl.ds(0, 3 * E)], sem_a)

            iota = lax.iota(jnp.int32, 16)
            moff = buf * (3 * E)

            def vreg2(v, carry):
                ne0, ne1 = carry
                base0 = v * 16
                base1 = (E // 2) + v * 16
                r0 = meta[pl.ds(moff + base0, 16)]
                r1 = meta[pl.ds(moff + base1, 16)]
                m0 = (r0 >= lo) & (r0 < hi)
                m1 = (r1 >= lo) & (r1 < hi)
                c0 = plsc.all_reduce_population_count(m0)[0]
                c1 = plsc.all_reduce_population_count(m1)[0]
                plsc.store_compressed(eix0.at[pl.ds(ne0, 16)],
                                      iota + base0, mask=m0)
                plsc.store_compressed(eix1.at[pl.ds(ne1, 16)],
                                      iota + base1, mask=m1)
                return (ne0 + c0, ne1 + c1)

            ne0, ne1 = lax.fori_loop(
                0, E // 32, vreg2, (jnp.int32(0), jnp.int32(0)), unroll=4)
            eix0[pl.ds(ne0, 16)] = jnp.zeros((16,), jnp.int32)
            eix1[pl.ds(ne1, 16)] = jnp.zeros((16,), jnp.int32)

            def expand(eix, nbase):
                def body(t, _):
                    ei = eix[pl.ds(16 * t, 16)] + moff
                    rr = plsc.load_gather(meta, [ei])
                    cv = plsc.load_gather(meta, [ei + E])
                    vv = plsc.load_gather(meta, [ei + 2 * E])
                    fl_r[pl.ds(nbase + 16 * t, 16)] = rr - lo
                    fl_c[pl.ds(nbase + 16 * t, 16)] = cv
                    fl_v[pl.ds(nbase + 16 * t, 16)] = plsc.bitcast(
                        vv, jnp.float32)
                    return 0
                return body

            lax.fori_loop(0, (ne0 + 15) // 16, expand(eix0, n), 0)
            lax.fori_loop(0, (ne1 + 15) // 16, expand(eix1, n + ne0), 0)
            n = n + ne0 + ne1
            return drain(n, cc, pend, ps0, ps1)

        n, cc, pend, ps0, ps1 = lax.fori_loop(
            0, NB, block, (jnp.int32(0), jnp.int32(0), jnp.int32(0),
                           jnp.int32(0), jnp.int32(0)))

        @pl.when(pend == 1)
        def _():
            complete_dyn((cc - 1) % 2)

        def pad(t, _):
            z = jnp.zeros((16,), jnp.int32)
            fl_r[pl.ds(n + 16 * t, 16)] = z
            fl_c[pl.ds(n + 16 * t, 16)] = z
            fl_v[pl.ds(n + 16 * t, 16)] = jnp.zeros((16,), jnp.float32)
            return 0

        lax.fori_loop(0, FC // 16, pad, 0)
        k_final = (n + FC - 1) // FC

        def jfinal(j, ps0):
            stage_issue(j, 0, ps0)
            complete(0)
            return jnp.int32(1)

        ps0 = lax.fori_loop(0, k_final, jfinal, ps0)

        @pl.when(ps0 == 1)
        def _():
            pltpu.make_async_copy(gbuf.at[0], acc.at[sidx.at[0]],
                                  semsc0).wait()

        @pl.when(ps1 == 1)
        def _():
            pltpu.make_async_copy(gbuf.at[1], acc.at[sidx.at[1]],
                                  semsc1).wait()

        plsc.subcore_barrier()
        pltpu.sync_copy(
            acc.at[pl.ds(s * ROWS_PER_TILE, ROWS_PER_TILE)],
            out_hbm.at[pl.ds(lo + s * ROWS_PER_TILE, ROWS_PER_TILE)])
        return 0

    np_c = jnp.where(c == 0, (NRANGES + 1) // 2, NRANGES // 2)
    lax.fori_loop(0, np_c, run_pass, 0)


@functools.partial(
    pl.kernel,
    out_type=jax.ShapeDtypeStruct((N1, D), jnp.float32),
    mesh=plsc.VectorSubcoreMesh(core_axis_name="c", subcore_axis_name="s"),
    compiler_params=pltpu.CompilerParams(needs_layout_passes=False),
    scratch_types=[
        pltpu.VMEM((2 * 3 * E,), jnp.int32),
        pltpu.VMEM((LISTCAP,), jnp.int32),
        pltpu.VMEM((LISTCAP,), jnp.int32),
        pltpu.VMEM((LISTCAP,), jnp.float32),
        pltpu.VMEM((352,), jnp.int32),
        pltpu.VMEM((352,), jnp.int32),
        pltpu.VMEM((2, FC), jnp.int32),
        pltpu.VMEM((2, FC), jnp.int32),
        pltpu.VMEM((2, FC), jnp.float32),
        pltpu.VMEM((2, FC, D), jnp.float32),
        pltpu.VMEM((ZROWS, D), jnp.float32),
        pltpu.MemorySpace.VMEM_SHARED((R, D), jnp.float32),
        pltpu.SemaphoreType.DMA,
        pltpu.SemaphoreType.DMA,
        pltpu.SemaphoreType.DMA,
        pltpu.SemaphoreType.DMA,
        pltpu.SemaphoreType.DMA,
        pltpu.SemaphoreType.DMA,
    ],
)
def _sc_spmm(packed_hbm, x_hbm, out_hbm, *scratch):
    _sc_spmm_body(packed_hbm, x_hbm, out_hbm, *scratch)


def _pack_meta(rows, cols, vals):
    nblk = NNZ // E
    vi = lax.bitcast_convert_type(vals, jnp.int32)
    return jnp.stack([rows.reshape(nblk, E), cols.reshape(nblk, E),
                      vi.reshape(nblk, E)], axis=1).reshape(nblk, 3 * E)


def _epilogue_body(s1_ref, s2_ref, x_ref, w_a_ref, w_b_ref, w_skip_ref,
                   g_ref, b_ref, out_ref):
    acc = jnp.dot(s1_ref[...], w_a_ref[...].T, preferred_element_type=jnp.float32)
    acc += jnp.dot(s2_ref[...], w_b_ref[...].T, preferred_element_type=jnp.float32)
    acc += jnp.dot(x_ref[...], w_skip_ref[...].T, preferred_element_type=jnp.float32)
    m = jnp.mean(acc, axis=-1, keepdims=True)
    cen = acc - m
    v = jnp.mean(cen * cen, axis=-1, keepdims=True)
    y = cen * jax.lax.rsqrt(v + 1e-5) * g_ref[...] + b_ref[...]
    out_ref[...] = 0.5 * y * (1.0 + jax.lax.erf(y * 0.7071067811865476))


def _epilogue(s1, s2, x, w_a, w_b, w_skip, g, b):
    grid = (N1 // BM,)
    blk = pl.BlockSpec((BM, D), lambda i: (i, 0))
    wblk = pl.BlockSpec((D, D), lambda i: (0, 0))
    vblk = pl.BlockSpec((1, D), lambda i: (0, 0))
    return pl.pallas_call(
        _epilogue_body,
        grid=grid,
        in_specs=[blk, blk, blk, wblk, wblk, wblk, vblk, vblk],
        out_specs=blk,
        out_shape=jax.ShapeDtypeStruct((N1, D), jnp.float32),
    )(s1, s2, x, w_a, w_b, w_skip, g.reshape(1, D), b.reshape(1, D))


def kernel(X_equ, X_inv, W1, W2, W3, W4, W5, W6, g_e, b_e, g_i, b_i,
           vals_Le, vals_ie, vals_Li, vals_ei,
           rows_Le, cols_Le, rows_ie, cols_ie, rows_Li, cols_Li,
           rows_ei, cols_ei):
    s_Le = _sc_spmm(_pack_meta(rows_Le, cols_Le, vals_Le), X_equ)
    s_ie = _sc_spmm(_pack_meta(rows_ie, cols_ie, vals_ie), X_inv)
    s_Li = _sc_spmm(_pack_meta(rows_Li, cols_Li, vals_Li), X_inv)
    s_ei = _sc_spmm(_pack_meta(rows_ei, cols_ei, vals_ei), X_equ)
    out_equ = _epilogue(s_Le, s_ie, X_equ, W1, W2, W5, g_e, b_e)
    out_inv = _epilogue(s_Li, s_ei, X_inv, W3, W4, W6, g_i, b_i)
    return (out_equ, out_inv)
